# baseline (speedup 1.0000x reference)
import jax
import jax.numpy as jnp
from jax.experimental import pallas as pl
from jax.experimental.pallas import tpu as pltpu

_IPS = 4


def _body(a1_ref, w1_ref, b1_ref, w2_ref, b2_ref, fcw_ref, fcb_ref,
          o_ref, r_ref, a2_ref):
    a1 = a1_ref[...].reshape(_IPS * 484, 144)
    z1 = jnp.dot(a1, w1_ref[...], preferred_element_type=jnp.float32)
    z1 = jnp.maximum(z1 + b1_ref[...], 0.0)

    ridx = jax.lax.broadcasted_iota(jnp.int32, (_IPS * 484, 1), 0) % 484
    ii, jj = ridx // 22, ridx % 22
    planes = []
    for q in range(4):
        s, u = q >> 1, q & 1
        pq = jnp.maximum(
            jnp.maximum(z1[:, 128 * q:128 * q + 32],
                        z1[:, 128 * q + 32:128 * q + 64]),
            jnp.maximum(z1[:, 128 * q + 64:128 * q + 96],
                        z1[:, 128 * q + 96:128 * q + 128]))
        oki = (ii >= 1) if s == 0 else (ii <= 20)
        okj = (jj >= 1) if u == 0 else (jj <= 20)
        planes.append(jnp.where(oki & okj, pq, 0.0))
    pooled = jnp.concatenate(planes, axis=1)

    for i in range(_IPS):
        r_ref[i, 0:484, :] = pooled[i * 484:(i + 1) * 484, :]
        r_ref[i, 484:488, :] = jnp.zeros((4, 128), jnp.float32)

    for u in range(4):
        for v in range(4):
            q = 2 * (u & 1) + (v & 1)
            row0 = (u >> 1) * 22 + (v >> 1)
            c0 = 32 * (4 * u + v)
            for i in range(_IPS):
                a2_ref[i * 462:(i + 1) * 462, c0:c0 + 32] = (
                    r_ref[i, row0:row0 + 462, 32 * q:32 * q + 32])

    z2 = jnp.dot(a2_ref[...], w2_ref[...],
                 preferred_element_type=jnp.float32)
    z2 = jnp.maximum(z2 + b2_ref[...], 0.0)
    p2 = jnp.maximum(jnp.maximum(z2[:, 0:64], z2[:, 64:128]),
                     jnp.maximum(z2[:, 128:192], z2[:, 192:256]))

    for i in range(_IPS):
        p2i = p2[i * 462:(i + 1) * 462, :]
        pr = jnp.concatenate(
            [jnp.sum(p2i * fcw_ref[o], axis=1, keepdims=True)
             for o in range(4)], axis=1)
        o_ref[i, 0:1, :] = (jnp.sum(pr, axis=0, keepdims=True)
                            + fcb_ref[...])


def _im2col_conv1(x_nchw):
    n = x_nchw.shape[0]
    xp = jnp.pad(x_nchw[:, 0].astype(jnp.float32), ((0, 0), (1, 1), (1, 1)))
    taps = []
    for ph in range(2):
        for pw in range(2):
            for dy in range(3):
                for dx in range(3):
                    taps.append(
                        xp[:, ph + dy:ph + dy + 84:2, pw + dx:pw + dx + 84:2])
    t = jnp.stack(taps, axis=-1)
    tp = jnp.pad(t, ((0, 0), (1, 1), (1, 1), (0, 0)))
    planes = jnp.stack([tp[:, s::2, u::2] for s in range(2) for u in range(2)],
                       axis=3)
    return planes.reshape(n, 484, 144).astype(jnp.bfloat16)


def _pack_params(conv1_w, conv1_b, conv2_w, conv2_b, fc_w, fc_b):
    w1 = jnp.kron(jnp.eye(16, dtype=jnp.float32),
                  conv1_w.reshape(32, 9).T).astype(jnp.bfloat16)
    b1 = jnp.tile(conv1_b, 16).reshape(1, 512)

    wt = jnp.transpose(conv2_w, (2, 3, 1, 0))
    cols = []
    for a in range(2):
        for b in range(2):
            m = jnp.zeros((4, 4, 32, 64), jnp.float32)
            m = m.at[a:a + 3, b:b + 3].set(wt)
            cols.append(m.reshape(512, 64))
    w2 = jnp.concatenate(cols, axis=1)
    b2 = jnp.tile(conv2_b, 4).reshape(1, 256)

    fw = jnp.transpose(fc_w.reshape(4, 64, 21, 21), (0, 2, 3, 1))
    fw = jnp.pad(fw, ((0, 0), (0, 0), (0, 1), (0, 0)))
    return w1, b1, w2, b2, fw.reshape(4, 462, 64), fc_b.reshape(1, 4)


@jax.jit
def _forward(x_nchw, conv1_w, conv1_b, conv2_w, conv2_b, fc_w, fc_b):
    n = x_nchw.shape[0]
    a1 = _im2col_conv1(x_nchw)
    w1, b1, w2, b2, fcw, fcb = _pack_params(
        conv1_w, conv1_b, conv2_w, conv2_b, fc_w, fc_b)

    out = pl.pallas_call(
        _body,
        out_shape=jax.ShapeDtypeStruct((n, 1, 4), jnp.float32),
        grid_spec=pltpu.PrefetchScalarGridSpec(
            num_scalar_prefetch=0,
            grid=(n // _IPS,),
            in_specs=[
                pl.BlockSpec((_IPS, 484, 144), lambda i: (i, 0, 0)),
                pl.BlockSpec((144, 512), lambda i: (0, 0)),
                pl.BlockSpec((1, 512), lambda i: (0, 0)),
                pl.BlockSpec((512, 256), lambda i: (0, 0)),
                pl.BlockSpec((1, 256), lambda i: (0, 0)),
                pl.BlockSpec((4, 462, 64), lambda i: (0, 0, 0)),
                pl.BlockSpec((1, 4), lambda i: (0, 0)),
            ],
            out_specs=pl.BlockSpec((_IPS, 1, 4), lambda i: (i, 0, 0)),
            scratch_shapes=[
                pltpu.VMEM((_IPS, 488, 128), jnp.float32),
                pltpu.VMEM((_IPS * 462, 512), jnp.float32),
            ],
        ),
        compiler_params=pltpu.CompilerParams(
            dimension_semantics=("parallel",),
            vmem_limit_bytes=64 * 1024 * 1024,
        ),
    )(a1, w1, b1, w2, b2, fcw, fcb)
    return out[:, 0, :]


def kernel(x_nchw, conv1_w, conv1_b, conv2_w, conv2_b, fc_w, fc_b):
    return _forward(x_nchw, conv1_w, conv1_b, conv2_w, conv2_b, fc_w, fc_b)


# speedup vs baseline: 2.1847x; 2.1847x over previous
import numpy as np
import jax
import jax.numpy as jnp
from jax.experimental import pallas as pl
from jax.experimental.pallas import tpu as pltpu

_IPS = 4


def _w1_tables():
    tap = np.zeros((64, 16), np.int32)
    valid = np.zeros((64, 16), np.float32)
    for row in range(64):
        a_coarse = row // 32 - 1
        b_coarse = (row // 16) % 2 - 1
        a4, b4 = (row // 4) % 4, row % 4
        aa, bb = 4 * a_coarse + a4, 4 * b_coarse + b4
        for q in range(4):
            s, u = q >> 1, q & 1
            for p in range(4):
                ph, pw = p >> 1, p & 1
                dy = aa - 2 * s - ph + 2
                dx = bb - 2 * u - pw + 2
                if 0 <= dy < 3 and 0 <= dx < 3:
                    tap[row, 4 * q + p] = 3 * dy + dx
                    valid[row, 4 * q + p] = 1.0
    return tap, valid


_TAP, _VALID = _w1_tables()


def _body(stk_ref, w1_ref, b1_ref, w2_ref, b2_ref, fcw_ref, fcb_ref,
          o_ref, aa_ref, r_ref, a2_ref):
    for i in range(_IPS):
        aa_ref[i * 484:(i + 1) * 484, 0:32] = stk_ref[i, 0:484, :]
        aa_ref[i * 484:(i + 1) * 484, 32:64] = stk_ref[i, 22:506, :]

    z1 = jnp.dot(aa_ref[...], w1_ref[...],
                 preferred_element_type=jnp.float32)
    z1 = jnp.maximum(z1 + b1_ref[...], 0.0)

    ridx = jax.lax.broadcasted_iota(jnp.int32, (_IPS * 484, 1), 0) % 484
    ii, jj = ridx // 22, ridx % 22
    planes = []
    for q in range(4):
        s, u = q >> 1, q & 1
        pq = jnp.maximum(
            jnp.maximum(z1[:, 128 * q:128 * q + 32],
                        z1[:, 128 * q + 32:128 * q + 64]),
            jnp.maximum(z1[:, 128 * q + 64:128 * q + 96],
                        z1[:, 128 * q + 96:128 * q + 128]))
        oki = (ii >= 1) if s == 0 else (ii <= 20)
        okj = (jj >= 1) if u == 0 else (jj <= 20)
        planes.append(jnp.where(oki & okj, pq, 0.0))
    pooled = jnp.concatenate(planes, axis=1)

    for i in range(_IPS):
        r_ref[i, 0:484, :] = pooled[i * 484:(i + 1) * 484, :]
        r_ref[i, 484:488, :] = jnp.zeros((4, 128), jnp.float32)

    for u in range(4):
        for v in range(4):
            q = 2 * (u & 1) + (v & 1)
            row0 = (u >> 1) * 22 + (v >> 1)
            c0 = 32 * (4 * u + v)
            for i in range(_IPS):
                a2_ref[i * 462:(i + 1) * 462, c0:c0 + 32] = (
                    r_ref[i, row0:row0 + 462, 32 * q:32 * q + 32]
                    .astype(jnp.bfloat16))

    z2 = jnp.dot(a2_ref[...], w2_ref[...],
                 preferred_element_type=jnp.float32)
    z2 = jnp.maximum(z2 + b2_ref[...], 0.0)
    p2 = jnp.maximum(jnp.maximum(z2[:, 0:64], z2[:, 64:128]),
                     jnp.maximum(z2[:, 128:192], z2[:, 192:256]))

    for i in range(_IPS):
        p2i = p2[i * 462:(i + 1) * 462, :]
        pr = jnp.concatenate(
            [jnp.sum(p2i * fcw_ref[o], axis=1, keepdims=True)
             for o in range(4)], axis=1)
        o_ref[i, 0:1, :] = (jnp.sum(pr, axis=0, keepdims=True)
                            + fcb_ref[...])


def _prep_stack(x_nchw):
    n = x_nchw.shape[0]
    xq = jnp.pad(x_nchw[:, 0].astype(jnp.float32), ((0, 0), (5, 7), (5, 7)))
    P = xq.reshape(n, 24, 4, 24, 4)
    pe = jnp.transpose(P[:, :, :, 0:22, :], (0, 1, 3, 2, 4)).reshape(n, 528, 16)
    po = jnp.transpose(P[:, :, :, 1:23, :], (0, 1, 3, 2, 4)).reshape(n, 528, 16)
    return jnp.concatenate([pe, po], axis=-1).astype(jnp.bfloat16)


def _pack_params(conv1_w, conv1_b, conv2_w, conv2_b, fc_w, fc_b):
    w9 = conv1_w.reshape(32, 9).T
    w1 = (w9[_TAP] * _VALID[:, :, None]).reshape(64, 512).astype(jnp.bfloat16)
    b1 = jnp.tile(conv1_b, 16).reshape(1, 512)

    wt = jnp.transpose(conv2_w, (2, 3, 1, 0))
    cols = []
    for a in range(2):
        for b in range(2):
            m = jnp.zeros((4, 4, 32, 64), jnp.float32)
            m = m.at[a:a + 3, b:b + 3].set(wt)
            cols.append(m.reshape(512, 64))
    w2 = jnp.concatenate(cols, axis=1).astype(jnp.bfloat16)
    b2 = jnp.tile(conv2_b, 4).reshape(1, 256)

    fw = jnp.transpose(fc_w.reshape(4, 64, 21, 21), (0, 2, 3, 1))
    fw = jnp.pad(fw, ((0, 0), (0, 0), (0, 1), (0, 0)))
    return w1, b1, w2, b2, fw.reshape(4, 462, 64), fc_b.reshape(1, 4)


@jax.jit
def _forward(x_nchw, conv1_w, conv1_b, conv2_w, conv2_b, fc_w, fc_b):
    n = x_nchw.shape[0]
    stk = _prep_stack(x_nchw)
    w1, b1, w2, b2, fcw, fcb = _pack_params(
        conv1_w, conv1_b, conv2_w, conv2_b, fc_w, fc_b)

    out = pl.pallas_call(
        _body,
        out_shape=jax.ShapeDtypeStruct((n, 1, 4), jnp.float32),
        grid_spec=pltpu.PrefetchScalarGridSpec(
            num_scalar_prefetch=0,
            grid=(n // _IPS,),
            in_specs=[
                pl.BlockSpec((_IPS, 528, 32), lambda i: (i, 0, 0)),
                pl.BlockSpec((64, 512), lambda i: (0, 0)),
                pl.BlockSpec((1, 512), lambda i: (0, 0)),
                pl.BlockSpec((512, 256), lambda i: (0, 0)),
                pl.BlockSpec((1, 256), lambda i: (0, 0)),
                pl.BlockSpec((4, 462, 64), lambda i: (0, 0, 0)),
                pl.BlockSpec((1, 4), lambda i: (0, 0)),
            ],
            out_specs=pl.BlockSpec((_IPS, 1, 4), lambda i: (i, 0, 0)),
            scratch_shapes=[
                pltpu.VMEM((_IPS * 484, 64), jnp.bfloat16),
                pltpu.VMEM((_IPS, 488, 128), jnp.float32),
                pltpu.VMEM((_IPS * 462, 512), jnp.bfloat16),
            ],
        ),
        compiler_params=pltpu.CompilerParams(
            dimension_semantics=("parallel",),
            vmem_limit_bytes=64 * 1024 * 1024,
        ),
    )(stk, w1, b1, w2, b2, fcw, fcb)
    return out[:, 0, :]


def kernel(x_nchw, conv1_w, conv1_b, conv2_w, conv2_b, fc_w, fc_b):
    return _forward(x_nchw, conv1_w, conv1_b, conv2_w, conv2_b, fc_w, fc_b)


# speedup vs baseline: 2.7341x; 1.2515x over previous
import numpy as np
import jax
import jax.numpy as jnp
from jax.experimental import pallas as pl
from jax.experimental.pallas import tpu as pltpu

_IPS = 4


def _w1_tables():
    tap = np.zeros((64, 16), np.int32)
    valid = np.zeros((64, 16), np.float32)
    for row in range(64):
        a_coarse = row // 32 - 1
        b_coarse = (row // 16) % 2 - 1
        a4, b4 = (row // 4) % 4, row % 4
        aa, bb = 4 * a_coarse + a4, 4 * b_coarse + b4
        for q in range(4):
            s, u = q >> 1, q & 1
            for p in range(4):
                ph, pw = p >> 1, p & 1
                dy = aa - 2 * s - ph + 2
                dx = bb - 2 * u - pw + 2
                if 0 <= dy < 3 and 0 <= dx < 3:
                    tap[row, 4 * q + p] = 3 * dy + dx
                    valid[row, 4 * q + p] = 1.0
    return tap, valid


_TAP, _VALID = _w1_tables()


def _body(stk_ref, w1_ref, b1_ref, w2s_ref, b2_ref, fcw_ref, fcb_ref,
          o_ref, aa_ref, r_ref):
    for i in range(_IPS):
        aa_ref[i * 484:(i + 1) * 484, 0:32] = stk_ref[i, 0:484, :]
        aa_ref[i * 484:(i + 1) * 484, 32:64] = stk_ref[i, 22:506, :]

    z1 = jnp.dot(aa_ref[...], w1_ref[...],
                 preferred_element_type=jnp.float32)
    z1 = jnp.maximum(z1 + b1_ref[...], 0.0)

    ridx = jax.lax.broadcasted_iota(jnp.int32, (_IPS * 484, 1), 0) % 484
    ii, jj = ridx // 22, ridx % 22
    planes = []
    for q in range(4):
        s, u = q >> 1, q & 1
        pq = jnp.maximum(
            jnp.maximum(z1[:, 128 * q:128 * q + 32],
                        z1[:, 128 * q + 32:128 * q + 64]),
            jnp.maximum(z1[:, 128 * q + 64:128 * q + 96],
                        z1[:, 128 * q + 96:128 * q + 128]))
        oki = (ii >= 1) if s == 0 else (ii <= 20)
        okj = (jj >= 1) if u == 0 else (jj <= 20)
        planes.append(jnp.where(oki & okj, pq, 0.0))
    pooled = jnp.concatenate(planes, axis=1)

    for i in range(_IPS):
        r_ref[i * 488:i * 488 + 484, :] = (
            pooled[i * 484:(i + 1) * 484, :].astype(jnp.bfloat16))
        r_ref[i * 488 + 484:(i + 1) * 488, :] = jnp.zeros(
            (4, 128), jnp.bfloat16)
    r_ref[_IPS * 488:, :] = jnp.zeros((24, 128), jnp.bfloat16)

    acc = None
    for k, row0 in enumerate((0, 1, 22, 23)):
        d = jnp.dot(r_ref[row0:row0 + _IPS * 488, :], w2s_ref[k],
                    preferred_element_type=jnp.float32)
        acc = d if acc is None else acc + d
    z2 = jnp.maximum(acc + b2_ref[...], 0.0)
    p2 = jnp.maximum(jnp.maximum(z2[:, 0:64], z2[:, 64:128]),
                     jnp.maximum(z2[:, 128:192], z2[:, 192:256]))

    for i in range(_IPS):
        p2i = p2[i * 488:i * 488 + 462, :]
        pr = jnp.concatenate(
            [jnp.sum(p2i * fcw_ref[o], axis=1, keepdims=True)
             for o in range(4)], axis=1)
        o_ref[i, 0:1, :] = (jnp.sum(pr, axis=0, keepdims=True)
                            + fcb_ref[...])


def _prep_stack(x_nchw):
    n = x_nchw.shape[0]
    xq = jnp.pad(x_nchw[:, 0].astype(jnp.float32), ((0, 0), (5, 7), (5, 7)))
    P = xq.reshape(n, 24, 4, 24, 4)
    pe = jnp.transpose(P[:, :, :, 0:22, :], (0, 1, 3, 2, 4)).reshape(n, 528, 16)
    po = jnp.transpose(P[:, :, :, 1:23, :], (0, 1, 3, 2, 4)).reshape(n, 528, 16)
    return jnp.concatenate([pe, po], axis=-1).astype(jnp.bfloat16)


def _pack_params(conv1_w, conv1_b, conv2_w, conv2_b, fc_w, fc_b):
    w9 = conv1_w.reshape(32, 9).T
    w1 = (w9[_TAP] * _VALID[:, :, None]).reshape(64, 512).astype(jnp.bfloat16)
    b1 = jnp.tile(conv1_b, 16).reshape(1, 512)

    wt = jnp.transpose(conv2_w, (2, 3, 1, 0))
    cols = []
    for a in range(2):
        for b in range(2):
            m = jnp.zeros((4, 4, 32, 64), jnp.float32)
            m = m.at[a:a + 3, b:b + 3].set(wt)
            cols.append(m.reshape(512, 64))
    w2big = jnp.concatenate(cols, axis=1)
    perm = np.zeros((4, 128), np.int32)
    for a in range(2):
        for b in range(2):
            for q in range(4):
                u, v = 2 * a + (q >> 1), 2 * b + (q & 1)
                perm[2 * a + b, 32 * q:32 * q + 32] = (
                    32 * (4 * u + v) + np.arange(32))
    w2s = w2big[perm].astype(jnp.bfloat16)
    b2 = jnp.tile(conv2_b, 4).reshape(1, 256)

    fw = jnp.transpose(fc_w.reshape(4, 64, 21, 21), (0, 2, 3, 1))
    fw = jnp.pad(fw, ((0, 0), (0, 0), (0, 1), (0, 0)))
    return w1, b1, w2s, b2, fw.reshape(4, 462, 64), fc_b.reshape(1, 4)


@jax.jit
def _forward(x_nchw, conv1_w, conv1_b, conv2_w, conv2_b, fc_w, fc_b):
    n = x_nchw.shape[0]
    stk = _prep_stack(x_nchw)
    w1, b1, w2s, b2, fcw, fcb = _pack_params(
        conv1_w, conv1_b, conv2_w, conv2_b, fc_w, fc_b)

    out = pl.pallas_call(
        _body,
        out_shape=jax.ShapeDtypeStruct((n, 1, 4), jnp.float32),
        grid_spec=pltpu.PrefetchScalarGridSpec(
            num_scalar_prefetch=0,
            grid=(n // _IPS,),
            in_specs=[
                pl.BlockSpec((_IPS, 528, 32), lambda i: (i, 0, 0)),
                pl.BlockSpec((64, 512), lambda i: (0, 0)),
                pl.BlockSpec((1, 512), lambda i: (0, 0)),
                pl.BlockSpec((4, 128, 256), lambda i: (0, 0, 0)),
                pl.BlockSpec((1, 256), lambda i: (0, 0)),
                pl.BlockSpec((4, 462, 64), lambda i: (0, 0, 0)),
                pl.BlockSpec((1, 4), lambda i: (0, 0)),
            ],
            out_specs=pl.BlockSpec((_IPS, 1, 4), lambda i: (i, 0, 0)),
            scratch_shapes=[
                pltpu.VMEM((_IPS * 484, 64), jnp.bfloat16),
                pltpu.VMEM((_IPS * 488 + 24, 128), jnp.bfloat16),
            ],
        ),
        compiler_params=pltpu.CompilerParams(
            dimension_semantics=("parallel",),
            vmem_limit_bytes=64 * 1024 * 1024,
        ),
    )(stk, w1, b1, w2s, b2, fcw, fcb)
    return out[:, 0, :]


def kernel(x_nchw, conv1_w, conv1_b, conv2_w, conv2_b, fc_w, fc_b):
    return _forward(x_nchw, conv1_w, conv1_b, conv2_w, conv2_b, fc_w, fc_b)


# speedup vs baseline: 3.2831x; 1.2008x over previous
import numpy as np
import jax
import jax.numpy as jnp
from jax.experimental import pallas as pl
from jax.experimental.pallas import tpu as pltpu

_IPS = 4


def _w1_tables():
    tap = np.zeros((64, 16), np.int32)
    valid = np.zeros((64, 16), np.float32)
    for row in range(64):
        a_coarse = row // 32 - 1
        b_coarse = (row // 16) % 2 - 1
        a4, b4 = (row // 4) % 4, row % 4
        aa, bb = 4 * a_coarse + a4, 4 * b_coarse + b4
        for q in range(4):
            s, u = q >> 1, q & 1
            for p in range(4):
                ph, pw = p >> 1, p & 1
                dy = aa - 2 * s - ph + 2
                dx = bb - 2 * u - pw + 2
                if 0 <= dy < 3 and 0 <= dx < 3:
                    tap[row, 4 * p + q] = 3 * dy + dx
                    valid[row, 4 * p + q] = 1.0
    return tap, valid


_TAP, _VALID = _w1_tables()


def _ring_mask():
    ii = np.arange(484)[:, None] // 22
    jj = np.arange(484)[:, None] % 22
    q = np.arange(128)[None, :] // 32
    oki = np.where(q < 2, ii >= 1, ii <= 20)
    okj = np.where(q % 2 == 0, jj >= 1, jj <= 20)
    m = np.zeros((488, 128), np.float32)
    m[:484] = (oki & okj).astype(np.float32)
    return m


_MASK = _ring_mask()


def _body(stk_ref, w1_ref, b1_ref, msk_ref, w2s_ref, b2_ref, fcw_ref,
          fcb_ref, o_ref, aa_ref, r_ref):
    for i in range(_IPS):
        aa_ref[i * 484:(i + 1) * 484, 0:32] = stk_ref[i, 0:484, :]
        aa_ref[i * 484:(i + 1) * 484, 32:64] = stk_ref[i, 22:506, :]

    z1 = jnp.dot(aa_ref[...], w1_ref[...],
                 preferred_element_type=jnp.float32)
    z1 = jnp.maximum(z1 + b1_ref[...], 0.0)

    pooled = jnp.maximum(jnp.maximum(z1[:, 0:128], z1[:, 128:256]),
                         jnp.maximum(z1[:, 256:384], z1[:, 384:512]))

    for i in range(_IPS):
        r_ref[i * 488:i * 488 + 484, :] = (
            (pooled[i * 484:(i + 1) * 484, :] * msk_ref[0:484, :])
            .astype(jnp.bfloat16))
        r_ref[i * 488 + 484:(i + 1) * 488, :] = jnp.zeros(
            (4, 128), jnp.bfloat16)
    r_ref[_IPS * 488:, :] = jnp.zeros((24, 128), jnp.bfloat16)

    acc = None
    for k, row0 in enumerate((0, 1, 22, 23)):
        d = jnp.dot(r_ref[row0:row0 + _IPS * 488, :], w2s_ref[k],
                    preferred_element_type=jnp.float32)
        acc = d if acc is None else acc + d
    z2 = jnp.maximum(acc + b2_ref[...], 0.0)
    m1 = jnp.maximum(z2[:, 0:128], z2[:, 128:256])
    p2 = jnp.maximum(m1[:, 0:64], m1[:, 64:128])

    for i in range(_IPS):
        p2i = p2[i * 488:i * 488 + 462, :]
        pr = jnp.concatenate(
            [jnp.sum(p2i * fcw_ref[o], axis=1, keepdims=True)
             for o in range(4)], axis=1)
        o_ref[i, 0:1, :] = (jnp.sum(pr, axis=0, keepdims=True)
                            + fcb_ref[...])


def _prep_stack(x_nchw):
    n = x_nchw.shape[0]
    xq = jnp.pad(x_nchw[:, 0].astype(jnp.float32), ((0, 0), (5, 7), (5, 7)))
    P = xq.reshape(n, 24, 4, 24, 4)
    pe = jnp.transpose(P[:, :, :, 0:22, :], (0, 1, 3, 2, 4)).reshape(n, 528, 16)
    po = jnp.transpose(P[:, :, :, 1:23, :], (0, 1, 3, 2, 4)).reshape(n, 528, 16)
    return jnp.concatenate([pe, po], axis=-1).astype(jnp.bfloat16)


def _pack_params(conv1_w, conv1_b, conv2_w, conv2_b, fc_w, fc_b):
    w9 = conv1_w.reshape(32, 9).T
    w1 = (w9[_TAP] * _VALID[:, :, None]).reshape(64, 512).astype(jnp.bfloat16)
    b1 = jnp.tile(conv1_b, 16).reshape(1, 512)

    wt = jnp.transpose(conv2_w, (2, 3, 1, 0))
    cols = []
    for a in range(2):
        for b in range(2):
            m = jnp.zeros((4, 4, 32, 64), jnp.float32)
            m = m.at[a:a + 3, b:b + 3].set(wt)
            cols.append(m.reshape(512, 64))
    w2big = jnp.concatenate(cols, axis=1)
    perm = np.zeros((4, 128), np.int32)
    for a in range(2):
        for b in range(2):
            for q in range(4):
                u, v = 2 * a + (q >> 1), 2 * b + (q & 1)
                perm[2 * a + b, 32 * q:32 * q + 32] = (
                    32 * (4 * u + v) + np.arange(32))
    w2s = w2big[perm].astype(jnp.bfloat16)
    b2 = jnp.tile(conv2_b, 4).reshape(1, 256)

    fw = jnp.transpose(fc_w.reshape(4, 64, 21, 21), (0, 2, 3, 1))
    fw = jnp.pad(fw, ((0, 0), (0, 0), (0, 1), (0, 0)))
    return w1, b1, w2s, b2, fw.reshape(4, 462, 64), fc_b.reshape(1, 4)


@jax.jit
def _forward(x_nchw, conv1_w, conv1_b, conv2_w, conv2_b, fc_w, fc_b):
    n = x_nchw.shape[0]
    stk = _prep_stack(x_nchw)
    w1, b1, w2s, b2, fcw, fcb = _pack_params(
        conv1_w, conv1_b, conv2_w, conv2_b, fc_w, fc_b)

    out = pl.pallas_call(
        _body,
        out_shape=jax.ShapeDtypeStruct((n, 1, 4), jnp.float32),
        grid_spec=pltpu.PrefetchScalarGridSpec(
            num_scalar_prefetch=0,
            grid=(n // _IPS,),
            in_specs=[
                pl.BlockSpec((_IPS, 528, 32), lambda i: (i, 0, 0)),
                pl.BlockSpec((64, 512), lambda i: (0, 0)),
                pl.BlockSpec((1, 512), lambda i: (0, 0)),
                pl.BlockSpec((488, 128), lambda i: (0, 0)),
                pl.BlockSpec((4, 128, 256), lambda i: (0, 0, 0)),
                pl.BlockSpec((1, 256), lambda i: (0, 0)),
                pl.BlockSpec((4, 462, 64), lambda i: (0, 0, 0)),
                pl.BlockSpec((1, 4), lambda i: (0, 0)),
            ],
            out_specs=pl.BlockSpec((_IPS, 1, 4), lambda i: (i, 0, 0)),
            scratch_shapes=[
                pltpu.VMEM((_IPS * 484, 64), jnp.bfloat16),
                pltpu.VMEM((_IPS * 488 + 24, 128), jnp.bfloat16),
            ],
        ),
        compiler_params=pltpu.CompilerParams(
            dimension_semantics=("parallel",),
            vmem_limit_bytes=64 * 1024 * 1024,
        ),
    )(stk, w1, b1, jnp.asarray(_MASK), w2s, b2, fcw, fcb)
    return out[:, 0, :]


def kernel(x_nchw, conv1_w, conv1_b, conv2_w, conv2_b, fc_w, fc_b):
    return _forward(x_nchw, conv1_w, conv1_b, conv2_w, conv2_b, fc_w, fc_b)
